# Initial kernel scaffold; baseline (speedup 1.0000x reference)
import jax
import jax.numpy as jnp
from jax.experimental import pallas as pl


def kernel(nodes, tmp_neighs, u2e_weight, att1_w, att1_b, att2_w, att2_b, att3_w, att3_b):
    raise NotImplementedError("write your pallas kernel here")


# speedup vs baseline: 1.9312x; 1.9312x over previous
import functools

import jax
import jax.numpy as jnp
from jax import lax
from jax.experimental import pallas as pl
from jax.experimental.pallas import tpu as pltpu
from jax.experimental.pallas import tpu_sc as plsc

NUM_USERS = 1000000
EMBED_DIM = 32
BATCH = 16384
DEG = 64

NC = 2
NS = 16
NW = NC * NS

N_EDGES = BATCH * DEG
EDGES_PER_W = N_EDGES // NW
NODES_PER_W = BATCH // NW
CHUNK = 1024
SUB = 128
N_CHUNKS = EDGES_PER_W // CHUNK


def _sc_gather_body(table_hbm, neigh_hbm, nodes_hbm, e_out, u_out,
                    idx_v, rows_v, idx_n, rows_n, sem):
    wid = lax.axis_index("s") * NC + lax.axis_index("c")

    pltpu.sync_copy(nodes_hbm.at[pl.ds(wid * (NODES_PER_W // SUB),
                                       NODES_PER_W // SUB)], idx_n)
    cps = []
    for j in range(NODES_PER_W // SUB):
        cps.append(pltpu.async_copy(
            table_hbm.at[idx_n.at[j]],
            rows_n.at[pl.ds(j * SUB, SUB)], sem))
    for cp in cps:
        cp.wait()
    pltpu.sync_copy(rows_n, u_out.at[pl.ds(wid * NODES_PER_W, NODES_PER_W)])

    idx_rows_per_chunk = CHUNK // SUB
    idx_row_base = wid * (EDGES_PER_W // SUB)
    row_base = wid * EDGES_PER_W

    def chunk_body(g, _):
        pltpu.sync_copy(
            neigh_hbm.at[pl.ds(idx_row_base + g * idx_rows_per_chunk,
                               idx_rows_per_chunk)], idx_v)
        cps = []
        for j in range(idx_rows_per_chunk):
            cps.append(pltpu.async_copy(
                table_hbm.at[idx_v.at[j]],
                rows_v.at[pl.ds(j * SUB, SUB)], sem))
        for cp in cps:
            cp.wait()
        pltpu.sync_copy(rows_v, e_out.at[pl.ds(row_base + g * CHUNK, CHUNK)])
        return 0

    lax.fori_loop(0, N_CHUNKS, chunk_body, 0)


def _sc_gather(table, neigh2, nodes2):
    mesh = plsc.VectorSubcoreMesh(core_axis_name="c", subcore_axis_name="s")
    k = pl.kernel(
        _sc_gather_body,
        out_type=(
            jax.ShapeDtypeStruct((N_EDGES, EMBED_DIM), jnp.float32),
            jax.ShapeDtypeStruct((BATCH, EMBED_DIM), jnp.float32),
        ),
        mesh=mesh,
        scratch_types=[
            pltpu.VMEM((CHUNK // SUB, SUB), jnp.int32),
            pltpu.VMEM((CHUNK, EMBED_DIM), jnp.float32),
            pltpu.VMEM((NODES_PER_W // SUB, SUB), jnp.int32),
            pltpu.VMEM((NODES_PER_W, EMBED_DIM), jnp.float32),
            pltpu.SemaphoreType.DMA,
        ],
        compiler_params=pltpu.CompilerParams(use_tc_tiling_on_sc=False),
    )
    return k(table, neigh2, nodes2)


def _tc_mlp_body(e_ref, u_ref, w1a_ref, w1b_ref, w2_ref, w3_ref,
                 b1_ref, b2_ref, o_ref, *, bB):
    e2 = e_ref[...]
    u = u_ref[...]
    r = jnp.dot(u, w1b_ref[...],
                preferred_element_type=jnp.float32) + b1_ref[...]
    t = jnp.dot(e2, w1a_ref[...], preferred_element_type=jnp.float32)
    h1 = jnp.maximum(t.reshape(bB, DEG, EMBED_DIM) + r[:, None, :], 0.0)
    h1 = h1.reshape(bB * DEG, EMBED_DIM)
    h2 = jnp.maximum(
        jnp.dot(h1, w2_ref[...], preferred_element_type=jnp.float32)
        + b2_ref[...], 0.0)
    w3 = w3_ref[...].reshape(1, 1, EMBED_DIM)
    s3 = jnp.sum(h2.reshape(bB, DEG, EMBED_DIM) * w3, axis=2,
                 keepdims=True)
    m = jnp.max(s3, axis=1, keepdims=True)
    p = jnp.exp(s3 - m)
    a3 = p / jnp.sum(p, axis=1, keepdims=True)
    e3 = e2.reshape(bB, DEG, EMBED_DIM)
    o_ref[...] = jnp.sum(a3 * e3, axis=1)


def _tc_mlp(e_u, u_rep, w1a, w1b, w2t, w3, b1, b2, bB=256):
    grid = (BATCH // bB,)
    return pl.pallas_call(
        functools.partial(_tc_mlp_body, bB=bB),
        grid=grid,
        in_specs=[
            pl.BlockSpec((bB * DEG, EMBED_DIM), lambda i: (i, 0)),
            pl.BlockSpec((bB, EMBED_DIM), lambda i: (i, 0)),
            pl.BlockSpec((EMBED_DIM, EMBED_DIM), lambda i: (0, 0)),
            pl.BlockSpec((EMBED_DIM, EMBED_DIM), lambda i: (0, 0)),
            pl.BlockSpec((EMBED_DIM, EMBED_DIM), lambda i: (0, 0)),
            pl.BlockSpec((1, EMBED_DIM), lambda i: (0, 0)),
            pl.BlockSpec((1, EMBED_DIM), lambda i: (0, 0)),
            pl.BlockSpec((1, EMBED_DIM), lambda i: (0, 0)),
        ],
        out_specs=pl.BlockSpec((bB, EMBED_DIM), lambda i: (i, 0)),
        out_shape=jax.ShapeDtypeStruct((BATCH, EMBED_DIM), jnp.float32),
    )(e_u, u_rep, w1a, w1b, w2t, w3, b1, b2)


def kernel(nodes, tmp_neighs, u2e_weight, att1_w, att1_b, att2_w, att2_b,
           att3_w, att3_b):
    del att3_b
    neigh2 = tmp_neighs.astype(jnp.int32).reshape(N_EDGES // SUB, SUB)
    nodes2 = nodes.astype(jnp.int32).reshape(BATCH // SUB, SUB)
    e_u, u_rep = _sc_gather(u2e_weight, neigh2, nodes2)
    w1t = att1_w.T
    w1a = w1t[:EMBED_DIM, :]
    w1b = w1t[EMBED_DIM:, :]
    w2t = att2_w.T
    b1 = att1_b.reshape(1, EMBED_DIM)
    b2 = att2_b.reshape(1, EMBED_DIM)
    return _tc_mlp(e_u, u_rep, w1a, w1b, w2t, att3_w, b1, b2)
